# baseline (speedup 1.0000x reference)
import functools

import jax
import jax.numpy as jnp
from jax import lax
from jax.experimental import pallas as pl
from jax.experimental.pallas import tpu as pltpu
from jax.experimental.pallas import tpu_sc as plsc

N_EV = 8192
N_T = 4096
SIZE = 1024
OUT_DIM = 512

NC, NS, L = 2, 16, 16
HALF = OUT_DIM // NC
EV_PER_TILE = N_EV // NS
CH = 64
NCH = EV_PER_TILE // CH
ROWS_PAD = N_T + 8


_CB = 512
_TB = 512


def _counts_body(times_ref, t_ref, out_ref):
    tc = times_ref[...]
    ones = jnp.ones((_TB, 8), jnp.bfloat16)

    parts = []
    for i in range(N_T // _TB):
        tr = t_ref[pl.ds(i, 1), :]
        m = (tr < tc).astype(jnp.bfloat16)
        parts.append(jax.lax.dot_general(
            m, ones, (((1,), (0,)), ((), ())),
            preferred_element_type=jnp.float32))
    acc = parts[0]
    for p in parts[1:]:
        acc = acc + p
    out_ref[...] = acc[:, :1].astype(jnp.int32)


def _counts(times_col, t2d):
    nblk = times_col.shape[0] // _CB
    return pl.pallas_call(
        _counts_body,
        grid=(nblk,),
        in_specs=[
            pl.BlockSpec((_CB, 1), lambda i: (i, 0)),
            pl.BlockSpec((8, _TB), lambda i: (0, 0)),
        ],
        out_specs=pl.BlockSpec((_CB, 1), lambda i: (i, 0)),
        out_shape=jax.ShapeDtypeStruct((times_col.shape[0], 1), jnp.int32),
    )(times_col, t2d)


CE = 1024
NCHK = N_EV // CE
ACC_ROWS = N_T + 8


def _sc_body(ks_hbm, ke_hbm, rate_hbm, idx_hbm, w_hbm, out_hbm,
             ks_v, ke_v, idx_v, rate_v, gpos, acc_v, sem):
    c = lax.axis_index("c")
    s = lax.axis_index("s")
    w = c * NS + s
    woff = jnp.full((L,), w, jnp.int32)

    def zrow(k8, _):
        for d in range(8):
            acc_v[k8 * 8 + d, :] = jnp.zeros((L,), jnp.float32)
        return 0
    lax.fori_loop(0, ACC_ROWS // 8, zrow, 0)

    def chunk(j, _):
        e0 = j * CE
        pltpu.sync_copy(ks_hbm.at[pl.ds(e0, CE)], ks_v)
        pltpu.sync_copy(ke_hbm.at[pl.ds(e0, CE)], ke_v)
        pltpu.sync_copy(idx_hbm.at[pl.ds(e0, CE)], idx_v)
        pltpu.sync_copy(rate_hbm.at[pl.ds(e0, CE)], rate_v)

        def adj(g, _):
            idx_v[pl.ds(g * L, L)] = idx_v[pl.ds(g * L, L)] + woff
            return 0
        lax.fori_loop(0, CE // L, adj, 0)

        pltpu.async_copy(w_hbm.at[idx_v], gpos, sem).wait()

        iota = lax.iota(jnp.int32, L)

        def scale_ev(i4, _):
            segs, rss, kbs, ebs = [], [], [], []
            for d in range(8):
                i = i4 * 8 + d
                bi = jnp.full((L,), i, jnp.int32)
                rss.append(plsc.load_gather(rate_v, [bi]))
                kbs.append(plsc.load_gather(ks_v, [bi]))
                ebs.append(plsc.load_gather(ke_v, [bi]))
                segs.append(gpos[i, :])
            for d in range(8):
                plsc.addupdate_scatter(acc_v, [kbs[d], iota], segs[d] * rss[d])
            for d in range(8):
                plsc.addupdate_scatter(acc_v, [ebs[d], iota],
                                       segs[d] * (-rss[d]))
            return 0
        lax.fori_loop(0, CE // 8, scale_ev, 0)
        return 0
    lax.fori_loop(0, NCHK, chunk, 0)

    def csum(k8, a):
        for d in range(8):
            a = a + acc_v[k8 * 8 + d, :]
            acc_v[k8 * 8 + d, :] = a
        return a
    lax.fori_loop(0, N_T // 8, csum, jnp.zeros((L,), jnp.float32))

    pltpu.sync_copy(acc_v.at[pl.ds(0, N_T)], out_hbm.at[:, pl.ds(w * L, L)])


@functools.cache
def _sc_call():
    return pl.kernel(
        _sc_body,
        out_type=jax.ShapeDtypeStruct((N_T, OUT_DIM), jnp.float32),
        mesh=plsc.VectorSubcoreMesh(core_axis_name="c", subcore_axis_name="s",
                                    num_cores=NC, num_subcores=NS),
        scratch_types=[
            pltpu.VMEM((CE,), jnp.int32),
            pltpu.VMEM((CE,), jnp.int32),
            pltpu.VMEM((CE,), jnp.int32),
            pltpu.VMEM((CE,), jnp.float32),
            pltpu.VMEM((CE, L), jnp.float32),
            pltpu.VMEM((ACC_ROWS, L), jnp.float32),
            pltpu.SemaphoreType.DMA,
        ],
        compiler_params=pltpu.CompilerParams(needs_layout_passes=False,
                                             use_tc_tiling_on_sc=False),
    )


def kernel(rate, starttime, endtime, t_query, W, index):
    times_col = jnp.concatenate([starttime, endtime]).reshape(-1, 1)
    t2d = t_query.reshape(8, N_T // 8)
    counts = _counts(times_col, t2d)[:, 0]

    ks = counts[:N_EV]
    ke = counts[N_EV:]
    idx = index.astype(jnp.int32) * (NC * NS)
    ws = W.reshape(SIZE * NC * NS, L)

    return _sc_call()(ks, ke, rate, idx, ws)


# speedup vs baseline: 1.0577x; 1.0577x over previous
import functools

import jax
import jax.numpy as jnp
from jax import lax
from jax.experimental import pallas as pl
from jax.experimental.pallas import tpu as pltpu
from jax.experimental.pallas import tpu_sc as plsc

N_EV = 8192
N_T = 4096
SIZE = 1024
OUT_DIM = 512

NC, NS, L = 2, 16, 16
HALF = OUT_DIM // NC
EV_PER_TILE = N_EV // NS
CH = 64
NCH = EV_PER_TILE // CH
ROWS_PAD = N_T + 8


_CB = 512
_TB = 512


def _counts_body(times_ref, t_ref, out_ref):
    tc = times_ref[...]
    ones = jnp.ones((_TB, 8), jnp.bfloat16)

    parts = []
    for i in range(N_T // _TB):
        tr = t_ref[pl.ds(i, 1), :]
        m = (tr < tc).astype(jnp.bfloat16)
        parts.append(jax.lax.dot_general(
            m, ones, (((1,), (0,)), ((), ())),
            preferred_element_type=jnp.float32))
    acc = parts[0]
    for p in parts[1:]:
        acc = acc + p
    out_ref[...] = acc[:, :1].astype(jnp.int32)


def _counts(times_col, t2d):
    nblk = times_col.shape[0] // _CB
    return pl.pallas_call(
        _counts_body,
        grid=(nblk,),
        in_specs=[
            pl.BlockSpec((_CB, 1), lambda i: (i, 0)),
            pl.BlockSpec((8, _TB), lambda i: (0, 0)),
        ],
        out_specs=pl.BlockSpec((_CB, 1), lambda i: (i, 0)),
        out_shape=jax.ShapeDtypeStruct((times_col.shape[0], 1), jnp.int32),
    )(times_col, t2d)


CE = 1024
NCHK = N_EV // CE
ACC_ROWS = N_T + 8


def _sc_body(kk_hbm, rate_hbm, idx_hbm, w_hbm, out_hbm,
             kk_v, idx_v, rate_v, gpos, acc_v, sem):
    c = lax.axis_index("c")
    s = lax.axis_index("s")
    w = c * NS + s
    woff = jnp.full((L,), w, jnp.int32)

    def zrow(k8, _):
        for d in range(8):
            acc_v[k8 * 8 + d, :] = jnp.zeros((L,), jnp.float32)
        return 0
    lax.fori_loop(0, ACC_ROWS // 8, zrow, 0)

    def chunk(j, _):
        e0 = j * CE
        pltpu.sync_copy(kk_hbm.at[pl.ds(e0, CE)], kk_v)
        pltpu.sync_copy(idx_hbm.at[pl.ds(e0, CE)], idx_v)
        pltpu.sync_copy(rate_hbm.at[pl.ds(e0, CE)], rate_v)

        def adj(g, _):
            idx_v[pl.ds(g * L, L)] = idx_v[pl.ds(g * L, L)] + woff
            return 0
        lax.fori_loop(0, CE // L, adj, 0)

        pltpu.async_copy(w_hbm.at[idx_v], gpos, sem).wait()

        iota = lax.iota(jnp.int32, L)

        def scale_ev(i4, _):
            segs, rss, kbs, ebs = [], [], [], []
            for d in range(8):
                i = i4 * 8 + d
                bi = jnp.full((L,), i, jnp.int32)
                rss.append(plsc.load_gather(rate_v, [bi]))
                kkb = plsc.load_gather(kk_v, [bi])
                kbs.append(kkb & 0x1FFF)
                ebs.append(kkb >> 13)
                segs.append(gpos[i, :])
            for d in range(8):
                plsc.addupdate_scatter(acc_v, [kbs[d], iota], segs[d] * rss[d])
            for d in range(8):
                plsc.addupdate_scatter(acc_v, [ebs[d], iota],
                                       segs[d] * (-rss[d]))
            return 0
        lax.fori_loop(0, CE // 8, scale_ev, 0)
        return 0
    lax.fori_loop(0, NCHK, chunk, 0)

    def csum(k8, a):
        for d in range(8):
            a = a + acc_v[k8 * 8 + d, :]
            acc_v[k8 * 8 + d, :] = a
        return a
    lax.fori_loop(0, N_T // 8, csum, jnp.zeros((L,), jnp.float32))

    pltpu.sync_copy(acc_v.at[pl.ds(0, N_T)], out_hbm.at[:, pl.ds(w * L, L)])


@functools.cache
def _sc_call():
    return pl.kernel(
        _sc_body,
        out_type=jax.ShapeDtypeStruct((N_T, OUT_DIM), jnp.float32),
        mesh=plsc.VectorSubcoreMesh(core_axis_name="c", subcore_axis_name="s",
                                    num_cores=NC, num_subcores=NS),
        scratch_types=[
            pltpu.VMEM((CE,), jnp.int32),
            pltpu.VMEM((CE,), jnp.int32),
            pltpu.VMEM((CE,), jnp.float32),
            pltpu.VMEM((CE, L), jnp.float32),
            pltpu.VMEM((ACC_ROWS, L), jnp.float32),
            pltpu.SemaphoreType.DMA,
        ],
        compiler_params=pltpu.CompilerParams(needs_layout_passes=False,
                                             use_tc_tiling_on_sc=False),
    )


def kernel(rate, starttime, endtime, t_query, W, index):
    times_col = jnp.concatenate([starttime, endtime]).reshape(-1, 1)
    t2d = t_query.reshape(8, N_T // 8)
    counts = _counts(times_col, t2d)[:, 0]

    kk = counts[:N_EV] + (counts[N_EV:] << 13)
    idx = index.astype(jnp.int32) * (NC * NS)
    ws = W.reshape(SIZE * NC * NS, L)

    return _sc_call()(kk, rate, idx, ws)


# speedup vs baseline: 1.1315x; 1.0697x over previous
import functools

import jax
import jax.numpy as jnp
from jax import lax
from jax.experimental import pallas as pl
from jax.experimental.pallas import tpu as pltpu
from jax.experimental.pallas import tpu_sc as plsc

N_EV = 8192
N_T = 4096
SIZE = 1024
OUT_DIM = 512

NC, NS, L = 2, 16, 16
HALF = OUT_DIM // NC
EV_PER_TILE = N_EV // NS
CH = 64
NCH = EV_PER_TILE // CH
ROWS_PAD = N_T + 8


_CB = 512
_TB = 512


def _counts_body(times_ref, t_ref, out_ref):
    tc = times_ref[...]
    ones = jnp.ones((_TB, 8), jnp.bfloat16)

    parts = []
    for i in range(N_T // _TB):
        tr = t_ref[pl.ds(i, 1), :]
        m = (tr < tc).astype(jnp.bfloat16)
        parts.append(jax.lax.dot_general(
            m, ones, (((1,), (0,)), ((), ())),
            preferred_element_type=jnp.float32))
    acc = parts[0]
    for p in parts[1:]:
        acc = acc + p
    out_ref[...] = acc[:, :1].astype(jnp.int32)


def _counts(times_col, t2d):
    nblk = times_col.shape[0] // _CB
    return pl.pallas_call(
        _counts_body,
        grid=(nblk,),
        in_specs=[
            pl.BlockSpec((_CB, 1), lambda i: (i, 0)),
            pl.BlockSpec((8, _TB), lambda i: (0, 0)),
        ],
        out_specs=pl.BlockSpec((_CB, 1), lambda i: (i, 0)),
        out_shape=jax.ShapeDtypeStruct((times_col.shape[0], 1), jnp.int32),
    )(times_col, t2d)


CE = 1024
NCHK = N_EV // CE
ACC_ROWS = N_T + 8


def _sc_body(kk_hbm, rate_hbm, idx_hbm, w_hbm, out_hbm,
             kk_v, idx_v, rate_v, gpos, acc_v, sem):
    c = lax.axis_index("c")
    s = lax.axis_index("s")
    w = c * NS + s
    woff = jnp.full((L,), w, jnp.int32)

    def zrow(k8, _):
        for d in range(8):
            acc_v[k8 * 8 + d, :] = jnp.zeros((L,), jnp.float32)
        return 0
    lax.fori_loop(0, ACC_ROWS // 8, zrow, 0)

    def load_meta(j, b0):
        e0 = j * CE
        pltpu.sync_copy(kk_hbm.at[pl.ds(e0, CE)], kk_v.at[pl.ds(b0, CE)])
        pltpu.sync_copy(idx_hbm.at[pl.ds(e0, CE)], idx_v.at[pl.ds(b0, CE)])
        pltpu.sync_copy(rate_hbm.at[pl.ds(e0, CE)], rate_v.at[pl.ds(b0, CE)])

        def adj(g, _):
            idx_v[pl.ds(b0 + g * L, L)] = idx_v[pl.ds(b0 + g * L, L)] + woff
            return 0
        lax.fori_loop(0, CE // L, adj, 0)

    def start_gather(b0):
        return pltpu.async_copy(
            w_hbm.at[idx_v.at[pl.ds(b0, CE)]],
            gpos.at[pl.ds(b0, CE), :], sem)

    load_meta(0, 0)
    start_gather(0)

    def chunk(j, _):
        b0 = (j % 2) * CE
        bn = CE - b0
        pltpu.make_async_copy(
            w_hbm.at[idx_v.at[pl.ds(b0, CE)]],
            gpos.at[pl.ds(b0, CE), :], sem).wait()

        @pl.when(j < NCHK - 1)
        def _():
            load_meta(j + 1, bn)
            start_gather(bn)

        iota = lax.iota(jnp.int32, L)

        def scale_ev(i4, _):
            segs, rss, kbs, ebs = [], [], [], []
            for d in range(8):
                i = b0 + i4 * 8 + d
                bi = jnp.full((L,), i, jnp.int32)
                rss.append(plsc.load_gather(rate_v, [bi]))
                kkb = plsc.load_gather(kk_v, [bi])
                kbs.append(kkb & 0x1FFF)
                ebs.append(kkb >> 13)
                segs.append(gpos[i, :])
            for d in range(8):
                plsc.addupdate_scatter(acc_v, [kbs[d], iota], segs[d] * rss[d])
            for d in range(8):
                plsc.addupdate_scatter(acc_v, [ebs[d], iota],
                                       segs[d] * (-rss[d]))
            return 0
        lax.fori_loop(0, CE // 8, scale_ev, 0)
        return 0
    lax.fori_loop(0, NCHK, chunk, 0)

    def csum(k8, a):
        for d in range(8):
            a = a + acc_v[k8 * 8 + d, :]
            acc_v[k8 * 8 + d, :] = a
        return a
    lax.fori_loop(0, N_T // 8, csum, jnp.zeros((L,), jnp.float32))

    pltpu.sync_copy(acc_v.at[pl.ds(0, N_T)], out_hbm.at[:, pl.ds(w * L, L)])


@functools.cache
def _sc_call():
    return pl.kernel(
        _sc_body,
        out_type=jax.ShapeDtypeStruct((N_T, OUT_DIM), jnp.float32),
        mesh=plsc.VectorSubcoreMesh(core_axis_name="c", subcore_axis_name="s",
                                    num_cores=NC, num_subcores=NS),
        scratch_types=[
            pltpu.VMEM((2 * CE,), jnp.int32),
            pltpu.VMEM((2 * CE,), jnp.int32),
            pltpu.VMEM((2 * CE,), jnp.float32),
            pltpu.VMEM((2 * CE, L), jnp.float32),
            pltpu.VMEM((ACC_ROWS, L), jnp.float32),
            pltpu.SemaphoreType.DMA,
        ],
        compiler_params=pltpu.CompilerParams(needs_layout_passes=False,
                                             use_tc_tiling_on_sc=False),
    )


def kernel(rate, starttime, endtime, t_query, W, index):
    times_col = jnp.concatenate([starttime, endtime]).reshape(-1, 1)
    t2d = t_query.reshape(8, N_T // 8)
    counts = _counts(times_col, t2d)[:, 0]

    kk = counts[:N_EV] + (counts[N_EV:] << 13)
    idx = index.astype(jnp.int32) * (NC * NS)
    ws = W.reshape(SIZE * NC * NS, L)

    return _sc_call()(kk, rate, idx, ws)


# speedup vs baseline: 1.1592x; 1.0245x over previous
import functools

import jax
import jax.numpy as jnp
from jax import lax
from jax.experimental import pallas as pl
from jax.experimental.pallas import tpu as pltpu
from jax.experimental.pallas import tpu_sc as plsc

N_EV = 8192
N_T = 4096
SIZE = 1024
OUT_DIM = 512

NC, NS, L = 2, 16, 16
HALF = OUT_DIM // NC
EV_PER_TILE = N_EV // NS
CH = 64
NCH = EV_PER_TILE // CH
ROWS_PAD = N_T + 8


_CB = 1024
_TB = 512


def _counts_body(times_ref, t_ref, out_ref):
    tc = times_ref[...]
    ones = jnp.ones((_TB, 8), jnp.bfloat16)

    parts = []
    for i in range(N_T // _TB):
        tr = t_ref[pl.ds(i, 1), :]
        m = (tr < tc).astype(jnp.bfloat16)
        parts.append(jax.lax.dot_general(
            m, ones, (((1,), (0,)), ((), ())),
            preferred_element_type=jnp.float32))
    acc = parts[0]
    for p in parts[1:]:
        acc = acc + p
    out_ref[...] = acc[:, :1].astype(jnp.int32)


def _counts(times_col, t2d):
    nblk = times_col.shape[0] // _CB
    return pl.pallas_call(
        _counts_body,
        grid=(nblk,),
        in_specs=[
            pl.BlockSpec((_CB, 1), lambda i: (i, 0)),
            pl.BlockSpec((8, _TB), lambda i: (0, 0)),
        ],
        out_specs=pl.BlockSpec((_CB, 1), lambda i: (i, 0)),
        out_shape=jax.ShapeDtypeStruct((times_col.shape[0], 1), jnp.int32),
    )(times_col, t2d)


CE = 1024
NCHK = N_EV // CE
ACC_ROWS = N_T + 8


def _sc_body(kk_hbm, rate_hbm, idx_hbm, w_hbm, out_hbm,
             kk_v, idx_v, rate_v, gpos, acc_v, sem):
    c = lax.axis_index("c")
    s = lax.axis_index("s")
    w = c * NS + s
    woff = jnp.full((L,), w, jnp.int32)

    def zrow(k8, _):
        for d in range(8):
            acc_v[k8 * 8 + d, :] = jnp.zeros((L,), jnp.float32)
        return 0
    lax.fori_loop(0, ACC_ROWS // 8, zrow, 0)

    def load_meta(j, b0):
        e0 = j * CE
        pltpu.sync_copy(kk_hbm.at[pl.ds(e0, CE)], kk_v.at[pl.ds(b0, CE)])
        pltpu.sync_copy(idx_hbm.at[pl.ds(e0, CE)], idx_v.at[pl.ds(b0, CE)])
        pltpu.sync_copy(rate_hbm.at[pl.ds(e0, CE)], rate_v.at[pl.ds(b0, CE)])

        def adj(g, _):
            idx_v[pl.ds(b0 + g * L, L)] = idx_v[pl.ds(b0 + g * L, L)] + woff
            return 0
        lax.fori_loop(0, CE // L, adj, 0)

    def start_gather(b0):
        return pltpu.async_copy(
            w_hbm.at[idx_v.at[pl.ds(b0, CE)]],
            gpos.at[pl.ds(b0, CE), :], sem)

    load_meta(0, 0)
    start_gather(0)

    def chunk(j, _):
        b0 = (j % 2) * CE
        bn = CE - b0
        pltpu.make_async_copy(
            w_hbm.at[idx_v.at[pl.ds(b0, CE)]],
            gpos.at[pl.ds(b0, CE), :], sem).wait()

        @pl.when(j < NCHK - 1)
        def _():
            load_meta(j + 1, bn)
            start_gather(bn)

        iota = lax.iota(jnp.int32, L)

        def scale_ev(i4, _):
            segs, rss, kbs, ebs = [], [], [], []
            for d in range(8):
                i = b0 + i4 * 8 + d
                bi = jnp.full((L,), i, jnp.int32)
                rss.append(plsc.load_gather(rate_v, [bi]))
                kkb = plsc.load_gather(kk_v, [bi])
                kbs.append(kkb & 0x1FFF)
                ebs.append(kkb >> 13)
                segs.append(gpos[i, :])
            for d in range(8):
                plsc.addupdate_scatter(acc_v, [kbs[d], iota], segs[d] * rss[d])
            for d in range(8):
                plsc.addupdate_scatter(acc_v, [ebs[d], iota],
                                       segs[d] * (-rss[d]))
            return 0
        lax.fori_loop(0, CE // 8, scale_ev, 0)
        return 0
    lax.fori_loop(0, NCHK, chunk, 0)

    def csum(k8, a):
        for d in range(8):
            a = a + acc_v[k8 * 8 + d, :]
            acc_v[k8 * 8 + d, :] = a
        return a
    lax.fori_loop(0, N_T // 8, csum, jnp.zeros((L,), jnp.float32))

    pltpu.sync_copy(acc_v.at[pl.ds(0, N_T)], out_hbm.at[:, pl.ds(w * L, L)])


@functools.cache
def _sc_call():
    return pl.kernel(
        _sc_body,
        out_type=jax.ShapeDtypeStruct((N_T, OUT_DIM), jnp.float32),
        mesh=plsc.VectorSubcoreMesh(core_axis_name="c", subcore_axis_name="s",
                                    num_cores=NC, num_subcores=NS),
        scratch_types=[
            pltpu.VMEM((2 * CE,), jnp.int32),
            pltpu.VMEM((2 * CE,), jnp.int32),
            pltpu.VMEM((2 * CE,), jnp.float32),
            pltpu.VMEM((2 * CE, L), jnp.float32),
            pltpu.VMEM((ACC_ROWS, L), jnp.float32),
            pltpu.SemaphoreType.DMA,
        ],
        compiler_params=pltpu.CompilerParams(needs_layout_passes=False,
                                             use_tc_tiling_on_sc=False),
    )


def kernel(rate, starttime, endtime, t_query, W, index):
    times_col = jnp.concatenate([starttime, endtime]).reshape(-1, 1)
    t2d = t_query.reshape(8, N_T // 8)
    counts = _counts(times_col, t2d)[:, 0]

    kk = counts[:N_EV] + (counts[N_EV:] << 13)
    idx = index.astype(jnp.int32) * (NC * NS)
    ws = W.reshape(SIZE * NC * NS, L)

    return _sc_call()(kk, rate, idx, ws)


# speedup vs baseline: 1.1700x; 1.0093x over previous
import functools

import jax
import jax.numpy as jnp
from jax import lax
from jax.experimental import pallas as pl
from jax.experimental.pallas import tpu as pltpu
from jax.experimental.pallas import tpu_sc as plsc

N_EV = 8192
N_T = 4096
SIZE = 1024
OUT_DIM = 512

NC, NS, L = 2, 16, 16
HALF = OUT_DIM // NC
EV_PER_TILE = N_EV // NS
CH = 64
NCH = EV_PER_TILE // CH
ROWS_PAD = N_T + 8


_CB = 2048
_TB = 512


def _counts_body(times_ref, t_ref, out_ref):
    tc = times_ref[...]
    ones = jnp.ones((_TB, 8), jnp.bfloat16)

    parts = []
    for i in range(N_T // _TB):
        tr = t_ref[pl.ds(i, 1), :]
        m = (tr < tc).astype(jnp.bfloat16)
        parts.append(jax.lax.dot_general(
            m, ones, (((1,), (0,)), ((), ())),
            preferred_element_type=jnp.float32))
    acc = parts[0]
    for p in parts[1:]:
        acc = acc + p
    out_ref[...] = acc[:, :1].astype(jnp.int32)


def _counts(times_col, t2d):
    nblk = times_col.shape[0] // _CB
    return pl.pallas_call(
        _counts_body,
        grid=(nblk,),
        in_specs=[
            pl.BlockSpec((_CB, 1), lambda i: (i, 0)),
            pl.BlockSpec((8, _TB), lambda i: (0, 0)),
        ],
        out_specs=pl.BlockSpec((_CB, 1), lambda i: (i, 0)),
        out_shape=jax.ShapeDtypeStruct((times_col.shape[0], 1), jnp.int32),
    )(times_col, t2d)


CE = 1024
NCHK = N_EV // CE
ACC_ROWS = N_T + 8


def _sc_body(kk_hbm, rate_hbm, idx_hbm, w_hbm, out_hbm,
             kk_v, idx_v, rate_v, gpos, acc_v, sem):
    c = lax.axis_index("c")
    s = lax.axis_index("s")
    w = c * NS + s
    woff = jnp.full((L,), w, jnp.int32)

    def zrow(k8, _):
        for d in range(8):
            acc_v[k8 * 8 + d, :] = jnp.zeros((L,), jnp.float32)
        return 0
    lax.fori_loop(0, ACC_ROWS // 8, zrow, 0)

    def load_meta(j, b0):
        e0 = j * CE
        pltpu.sync_copy(kk_hbm.at[pl.ds(e0, CE)], kk_v.at[pl.ds(b0, CE)])
        pltpu.sync_copy(idx_hbm.at[pl.ds(e0, CE)], idx_v.at[pl.ds(b0, CE)])
        pltpu.sync_copy(rate_hbm.at[pl.ds(e0, CE)], rate_v.at[pl.ds(b0, CE)])

        def adj(g, _):
            idx_v[pl.ds(b0 + g * L, L)] = idx_v[pl.ds(b0 + g * L, L)] + woff
            return 0
        lax.fori_loop(0, CE // L, adj, 0)

    def start_gather(b0):
        return pltpu.async_copy(
            w_hbm.at[idx_v.at[pl.ds(b0, CE)]],
            gpos.at[pl.ds(b0, CE), :], sem)

    load_meta(0, 0)
    start_gather(0)

    def chunk(j, _):
        b0 = (j % 2) * CE
        bn = CE - b0
        pltpu.make_async_copy(
            w_hbm.at[idx_v.at[pl.ds(b0, CE)]],
            gpos.at[pl.ds(b0, CE), :], sem).wait()

        @pl.when(j < NCHK - 1)
        def _():
            load_meta(j + 1, bn)
            start_gather(bn)

        iota = lax.iota(jnp.int32, L)

        def scale_ev(i4, _):
            segs, rss, kbs, ebs = [], [], [], []
            for d in range(8):
                i = b0 + i4 * 8 + d
                bi = jnp.full((L,), i, jnp.int32)
                rss.append(plsc.load_gather(rate_v, [bi]))
                kkb = plsc.load_gather(kk_v, [bi])
                kbs.append(kkb & 0x1FFF)
                ebs.append(kkb >> 13)
                segs.append(gpos[i, :])
            for d in range(8):
                plsc.addupdate_scatter(acc_v, [kbs[d], iota], segs[d] * rss[d])
            for d in range(8):
                plsc.addupdate_scatter(acc_v, [ebs[d], iota],
                                       segs[d] * (-rss[d]))
            return 0
        lax.fori_loop(0, CE // 8, scale_ev, 0)
        return 0
    lax.fori_loop(0, NCHK, chunk, 0)

    def csum(k8, a):
        for d in range(8):
            a = a + acc_v[k8 * 8 + d, :]
            acc_v[k8 * 8 + d, :] = a
        return a
    lax.fori_loop(0, N_T // 8, csum, jnp.zeros((L,), jnp.float32))

    pltpu.sync_copy(acc_v.at[pl.ds(0, N_T)], out_hbm.at[:, pl.ds(w * L, L)])


@functools.cache
def _sc_call():
    return pl.kernel(
        _sc_body,
        out_type=jax.ShapeDtypeStruct((N_T, OUT_DIM), jnp.float32),
        mesh=plsc.VectorSubcoreMesh(core_axis_name="c", subcore_axis_name="s",
                                    num_cores=NC, num_subcores=NS),
        scratch_types=[
            pltpu.VMEM((2 * CE,), jnp.int32),
            pltpu.VMEM((2 * CE,), jnp.int32),
            pltpu.VMEM((2 * CE,), jnp.float32),
            pltpu.VMEM((2 * CE, L), jnp.float32),
            pltpu.VMEM((ACC_ROWS, L), jnp.float32),
            pltpu.SemaphoreType.DMA,
        ],
        compiler_params=pltpu.CompilerParams(needs_layout_passes=False,
                                             use_tc_tiling_on_sc=False),
    )


def kernel(rate, starttime, endtime, t_query, W, index):
    times_col = jnp.concatenate([starttime, endtime]).reshape(-1, 1)
    t2d = t_query.reshape(8, N_T // 8)
    counts = _counts(times_col, t2d)[:, 0]

    kk = counts[:N_EV] + (counts[N_EV:] << 13)
    idx = index.astype(jnp.int32) * (NC * NS)
    ws = W.reshape(SIZE * NC * NS, L)

    return _sc_call()(kk, rate, idx, ws)
